# Initial kernel scaffold; baseline (speedup 1.0000x reference)
import functools

import jax
import jax.numpy as jnp
from jax import lax
from jax.experimental import pallas as pl
from jax.experimental.pallas import tpu as pltpu
from jax.experimental.pallas import tpu_sc as plsc

N_NODES = 10000
N_EDGES = 320000
D = 128

NC = 2
NS = 16
NW = NC * NS

E_PER_TILE = 10240
E_PAD = E_PER_TILE * NW
IDX_ROWS = E_PAD // 128
ROWS_PER_TILE = IDX_ROWS // NW
BLOCKS_PER_TILE = ROWS_PER_TILE // 8

H_ROWS = N_NODES + 16
ACC_ROWS = 10240

_mesh = plsc.VectorSubcoreMesh(core_axis_name="c", subcore_axis_name="s")


def _worker_id():
    return lax.axis_index("c") * NS + lax.axis_index("s")


@functools.partial(
    pl.kernel,
    out_type=jax.ShapeDtypeStruct((NC, ACC_ROWS), jnp.float32),
    mesh=_mesh,
    scratch_types=[
        pltpu.VMEM((8, 128), jnp.int32),
        pltpu.VMEM((128,), jnp.float32),
        pltpu.VMEM((640,), jnp.float32),
        pltpu.VMEM_SHARED((ACC_ROWS,), jnp.float32),
    ],
)
def _sc_degree(dst_hbm, out_hbm, idx_v, ones_v, z_v, acc_sh):
    cid = lax.axis_index("c")
    sid = lax.axis_index("s")
    wid = _worker_id()

    one16 = jnp.ones((16,), jnp.float32)
    zero16 = jnp.zeros((16,), jnp.float32)
    for j in range(8):
        ones_v[pl.ds(j * 16, 16)] = one16
    for j in range(40):
        z_v[pl.ds(j * 16, 16)] = zero16

    pltpu.sync_copy(z_v, acc_sh.at[pl.ds(sid * 640, 640)])
    plsc.subcore_barrier()

    def blk(b, carry):
        base = wid * ROWS_PER_TILE + b * 8
        pltpu.sync_copy(dst_hbm.at[pl.ds(base, 8)], idx_v)
        for j in range(8):
            pltpu.sync_copy(ones_v, acc_sh.at[idx_v.at[j]], add=True)
        return carry

    lax.fori_loop(0, BLOCKS_PER_TILE, blk, 0)
    plsc.subcore_barrier()
    pltpu.sync_copy(acc_sh.at[pl.ds(sid * 640, 640)],
                    out_hbm.at[cid, pl.ds(sid * 640, 640)])


@functools.partial(
    pl.kernel,
    out_type=jax.ShapeDtypeStruct((NC, ACC_ROWS, D), jnp.float32),
    mesh=_mesh,
    scratch_types=[
        pltpu.VMEM((8, 128), jnp.int32),
        pltpu.VMEM((8, 128), jnp.int32),
        pltpu.VMEM((128, D), jnp.float32),
        pltpu.VMEM((16, D), jnp.float32),
        pltpu.VMEM_SHARED((ACC_ROWS, D), jnp.float32),
        pltpu.SemaphoreType.DMA,
    ],
)
def _sc_aggregate(h_hbm, src_hbm, dst_hbm, out_hbm,
                  sidx_v, didx_v, rows_v, z_v, acc_sh, sem):
    cid = lax.axis_index("c")
    sid = lax.axis_index("s")
    wid = _worker_id()

    zero16 = jnp.zeros((16,), jnp.float32)
    for r in range(16):
        for c in range(8):
            z_v[r, pl.ds(c * 16, 16)] = zero16

    def zcp(t, carry):
        pltpu.sync_copy(z_v, acc_sh.at[pl.ds(sid * 640 + t * 16, 16)])
        return carry

    lax.fori_loop(0, 40, zcp, 0)
    plsc.subcore_barrier()

    def blk(b, carry):
        base = wid * ROWS_PER_TILE + b * 8
        pltpu.sync_copy(src_hbm.at[pl.ds(base, 8)], sidx_v)
        pltpu.sync_copy(dst_hbm.at[pl.ds(base, 8)], didx_v)
        for j in range(8):
            pltpu.async_copy(h_hbm.at[sidx_v.at[j]], rows_v, sem).wait()
            pltpu.sync_copy(rows_v, acc_sh.at[didx_v.at[j]], add=True)
        return carry

    lax.fori_loop(0, BLOCKS_PER_TILE, blk, 0)
    plsc.subcore_barrier()
    pltpu.sync_copy(acc_sh.at[pl.ds(sid * 640, 640)],
                    out_hbm.at[cid, pl.ds(sid * 640, 640)])


_GRID = 10
_BR = N_NODES // _GRID


def _dis_of(degp_ref):
    deg = degp_ref[0, :] + degp_ref[1, :] + 1.0
    return lax.rsqrt(deg)


def _tc1_body(x_ref, w_ref, degp_ref, o_ref):
    dis = _dis_of(degp_ref)
    h = jnp.dot(x_ref[...], w_ref[...], preferred_element_type=jnp.float32)
    o_ref[...] = h * dis[:, None]


def _tc2_body(agg_ref, hp_ref, degp_ref, b_ref, w_ref, o_ref):
    dis = _dis_of(degp_ref)
    t = (agg_ref[0] + agg_ref[1] + hp_ref[...]) * dis[:, None] + b_ref[...]
    t = jnp.maximum(t, 0.0)
    h = jnp.dot(t, w_ref[...], preferred_element_type=jnp.float32)
    o_ref[...] = h * dis[:, None]


def _tc3_body(agg_ref, hp_ref, degp_ref, b_ref, o_ref):
    dis = _dis_of(degp_ref)
    o_ref[...] = ((agg_ref[0] + agg_ref[1] + hp_ref[...]) * dis[:, None]
                  + b_ref[...])


_ROWS_SPEC = pl.BlockSpec((_BR, D), lambda i: (i, 0))
_W_SPEC = pl.BlockSpec((D, D), lambda i: (0, 0))
_DEG_SPEC = pl.BlockSpec((NC, _BR), lambda i: (0, i))
_AGG_SPEC = pl.BlockSpec((NC, _BR, D), lambda i: (0, i, 0))
_B_SPEC = pl.BlockSpec((1, D), lambda i: (0, 0))

_tc1 = pl.pallas_call(
    _tc1_body,
    grid=(_GRID,),
    in_specs=[_ROWS_SPEC, _W_SPEC, _DEG_SPEC],
    out_specs=_ROWS_SPEC,
    out_shape=jax.ShapeDtypeStruct((N_NODES, D), jnp.float32),
)

_tc2 = pl.pallas_call(
    _tc2_body,
    grid=(_GRID,),
    in_specs=[_AGG_SPEC, _ROWS_SPEC, _DEG_SPEC, _B_SPEC, _W_SPEC],
    out_specs=_ROWS_SPEC,
    out_shape=jax.ShapeDtypeStruct((N_NODES, D), jnp.float32),
)

_tc3 = pl.pallas_call(
    _tc3_body,
    grid=(_GRID,),
    in_specs=[_AGG_SPEC, _ROWS_SPEC, _DEG_SPEC, _B_SPEC],
    out_specs=_ROWS_SPEC,
    out_shape=jax.ShapeDtypeStruct((N_NODES, D), jnp.float32),
)


def _pad_h(h):
    return jnp.concatenate([h, jnp.zeros((H_ROWS - N_NODES, D), jnp.float32)],
                           axis=0)


def kernel(x, edge_index, W1, b1, W2, b2):
    src = edge_index[0].astype(jnp.int32)
    dst = edge_index[1].astype(jnp.int32)

    npad = E_PAD - N_EDGES
    pad_i = jnp.arange(npad, dtype=jnp.int32)
    src_p = jnp.concatenate([src, N_NODES + pad_i % (H_ROWS - N_NODES)])
    dst_p = jnp.concatenate([dst, N_NODES + pad_i % (ACC_ROWS - N_NODES)])
    src2d = src_p.reshape(IDX_ROWS, 128)
    dst2d = dst_p.reshape(IDX_ROWS, 128)

    b1r = b1.reshape(1, D)
    b2r = b2.reshape(1, D)

    degp = _sc_degree(dst2d)
    h1 = _tc1(x, W1, degp)
    agg1 = _sc_aggregate(_pad_h(h1), src2d, dst2d)
    h2 = _tc2(agg1, h1, degp, b1r, W2)
    agg2 = _sc_aggregate(_pad_h(h2), src2d, dst2d)
    return _tc3(agg2, h2, degp, b2r)


# speedup vs baseline: 21.2765x; 21.2765x over previous
import functools

import jax
import jax.numpy as jnp
from jax import lax
from jax.experimental import pallas as pl
from jax.experimental.pallas import tpu as pltpu
from jax.experimental.pallas import tpu_sc as plsc

N_NODES = 10000
N_EDGES = 320000
D = 128

NC = 2
NS = 16
NW = NC * NS

E_PER_TILE = 10240
E_PAD = E_PER_TILE * NW
IDX_ROWS = E_PAD // 128
ROWS_PER_TILE = IDX_ROWS // NW
BLOCKS_PER_TILE = ROWS_PER_TILE // 8

H_ROWS = N_NODES + 16
ACC_ROWS = 10240

_mesh = plsc.VectorSubcoreMesh(core_axis_name="c", subcore_axis_name="s")


def _worker_id():
    return lax.axis_index("c") * NS + lax.axis_index("s")


@functools.partial(
    pl.kernel,
    out_type=jax.ShapeDtypeStruct((NC, ACC_ROWS), jnp.float32),
    mesh=_mesh,
    scratch_types=[
        pltpu.VMEM((8, 128), jnp.int32),
        pltpu.VMEM((128,), jnp.float32),
        pltpu.VMEM((640,), jnp.float32),
        pltpu.VMEM_SHARED((ACC_ROWS,), jnp.float32),
    ],
)
def _sc_degree(dst_hbm, out_hbm, idx_v, ones_v, z_v, acc_sh):
    cid = lax.axis_index("c")
    sid = lax.axis_index("s")
    wid = _worker_id()

    one16 = jnp.ones((16,), jnp.float32)
    zero16 = jnp.zeros((16,), jnp.float32)
    for j in range(8):
        ones_v[pl.ds(j * 16, 16)] = one16
    for j in range(40):
        z_v[pl.ds(j * 16, 16)] = zero16

    pltpu.sync_copy(z_v, acc_sh.at[pl.ds(sid * 640, 640)])
    plsc.subcore_barrier()

    def blk(b, carry):
        base = wid * ROWS_PER_TILE + b * 8
        pltpu.sync_copy(dst_hbm.at[pl.ds(base, 8)], idx_v)
        for j in range(8):
            pltpu.sync_copy(ones_v, acc_sh.at[idx_v.at[j]], add=True)
        return carry

    lax.fori_loop(0, BLOCKS_PER_TILE, blk, 0)
    plsc.subcore_barrier()
    pltpu.sync_copy(acc_sh.at[pl.ds(sid * 640, 640)],
                    out_hbm.at[cid, pl.ds(sid * 640, 640)])


@functools.partial(
    pl.kernel,
    out_type=jax.ShapeDtypeStruct((NC, ACC_ROWS, D), jnp.float32),
    mesh=_mesh,
    scratch_types=[
        pltpu.VMEM((8, 128), jnp.int32),
        pltpu.VMEM((8, 128), jnp.int32),
        pltpu.VMEM((128, D), jnp.float32),
        pltpu.VMEM((16, D), jnp.float32),
        pltpu.VMEM_SHARED((ACC_ROWS, D), jnp.float32),
        pltpu.SemaphoreType.DMA,
    ],
)
def _sc_aggregate(h_hbm, src_hbm, dst_hbm, out_hbm,
                  sidx_v, didx_v, rows_v, z_v, acc_sh, sem):
    cid = lax.axis_index("c")
    sid = lax.axis_index("s")
    wid = _worker_id()

    zero16 = jnp.zeros((16,), jnp.float32)
    for r in range(16):
        for c in range(8):
            z_v[r, pl.ds(c * 16, 16)] = zero16

    def zcp(t, carry):
        pltpu.sync_copy(z_v, acc_sh.at[pl.ds(sid * 640 + t * 16, 16)])
        return carry

    lax.fori_loop(0, 40, zcp, 0)
    plsc.subcore_barrier()

    def blk(b, carry):
        base = wid * ROWS_PER_TILE + b * 8
        pltpu.sync_copy(src_hbm.at[pl.ds(base, 8)], sidx_v)
        pltpu.sync_copy(dst_hbm.at[pl.ds(base, 8)], didx_v)
        for j in range(8):
            pltpu.async_copy(h_hbm.at[sidx_v.at[j]], rows_v, sem).wait()
            pltpu.sync_copy(rows_v, acc_sh.at[didx_v.at[j]], add=True)
        return carry

    lax.fori_loop(0, BLOCKS_PER_TILE, blk, 0)
    plsc.subcore_barrier()
    pltpu.sync_copy(acc_sh.at[pl.ds(sid * 640, 640)],
                    out_hbm.at[cid, pl.ds(sid * 640, 640)])


_GRID = 10
_BR = N_NODES // _GRID


def _dis_of(degp_ref):
    deg = degp_ref[:, 0] + degp_ref[:, 1] + 1.0
    return lax.rsqrt(deg)


def _tc1_body(x_ref, w_ref, degp_ref, o_ref):
    dis = _dis_of(degp_ref)
    h = jnp.dot(x_ref[...], w_ref[...], preferred_element_type=jnp.float32)
    o_ref[...] = h * dis[:, None]


def _tc2_body(agg_ref, hp_ref, degp_ref, b_ref, w_ref, o_ref):
    dis = _dis_of(degp_ref)
    t = (agg_ref[0] + agg_ref[1] + hp_ref[...]) * dis[:, None] + b_ref[...]
    t = jnp.maximum(t, 0.0)
    h = jnp.dot(t, w_ref[...], preferred_element_type=jnp.float32)
    o_ref[...] = h * dis[:, None]


def _tc3_body(agg_ref, hp_ref, degp_ref, b_ref, o_ref):
    dis = _dis_of(degp_ref)
    o_ref[...] = ((agg_ref[0] + agg_ref[1] + hp_ref[...]) * dis[:, None]
                  + b_ref[...])


_ROWS_SPEC = pl.BlockSpec((_BR, D), lambda i: (i, 0))
_W_SPEC = pl.BlockSpec((D, D), lambda i: (0, 0))
_DEG_SPEC = pl.BlockSpec((_BR, NC), lambda i: (i, 0))
_AGG_SPEC = pl.BlockSpec((NC, _BR, D), lambda i: (0, i, 0))
_B_SPEC = pl.BlockSpec((1, D), lambda i: (0, 0))

_tc1 = pl.pallas_call(
    _tc1_body,
    grid=(_GRID,),
    in_specs=[_ROWS_SPEC, _W_SPEC, _DEG_SPEC],
    out_specs=_ROWS_SPEC,
    out_shape=jax.ShapeDtypeStruct((N_NODES, D), jnp.float32),
)

_tc2 = pl.pallas_call(
    _tc2_body,
    grid=(_GRID,),
    in_specs=[_AGG_SPEC, _ROWS_SPEC, _DEG_SPEC, _B_SPEC, _W_SPEC],
    out_specs=_ROWS_SPEC,
    out_shape=jax.ShapeDtypeStruct((N_NODES, D), jnp.float32),
)

_tc3 = pl.pallas_call(
    _tc3_body,
    grid=(_GRID,),
    in_specs=[_AGG_SPEC, _ROWS_SPEC, _DEG_SPEC, _B_SPEC],
    out_specs=_ROWS_SPEC,
    out_shape=jax.ShapeDtypeStruct((N_NODES, D), jnp.float32),
)


def _pad_h(h):
    return jnp.concatenate([h, jnp.zeros((H_ROWS - N_NODES, D), jnp.float32)],
                           axis=0)


def kernel(x, edge_index, W1, b1, W2, b2):
    src = edge_index[0].astype(jnp.int32)
    dst = edge_index[1].astype(jnp.int32)

    npad = E_PAD - N_EDGES
    pad_i = jnp.arange(npad, dtype=jnp.int32)
    src_p = jnp.concatenate([src, N_NODES + pad_i % (H_ROWS - N_NODES)])
    dst_p = jnp.concatenate([dst, N_NODES + pad_i % (ACC_ROWS - N_NODES)])
    src2d = src_p.reshape(IDX_ROWS, 128)
    dst2d = dst_p.reshape(IDX_ROWS, 128)

    b1r = b1.reshape(1, D)
    b2r = b2.reshape(1, D)

    degp = _sc_degree(dst2d).T
    h1 = _tc1(x, W1, degp)
    agg1 = _sc_aggregate(_pad_h(h1), src2d, dst2d)
    h2 = _tc2(agg1, h1, degp, b1r, W2)
    agg2 = _sc_aggregate(_pad_h(h2), src2d, dst2d)
    return _tc3(agg2, h2, degp, b2r)


# speedup vs baseline: 32.9519x; 1.5488x over previous
import functools

import jax
import jax.numpy as jnp
from jax import lax
from jax.experimental import pallas as pl
from jax.experimental.pallas import tpu as pltpu
from jax.experimental.pallas import tpu_sc as plsc

N_NODES = 10000
N_EDGES = 320000
D = 128

NC = 2
NS = 16
NW = NC * NS

E_PER_TILE = 10240
E_PAD = E_PER_TILE * NW
IDX_ROWS = E_PAD // 128
ROWS_PER_TILE = IDX_ROWS // NW
BLOCKS_PER_TILE = ROWS_PER_TILE // 8

H_ROWS = N_NODES + 16
ACC_ROWS = 10240

_mesh = plsc.VectorSubcoreMesh(core_axis_name="c", subcore_axis_name="s")


def _worker_id():
    return lax.axis_index("c") * NS + lax.axis_index("s")


@functools.partial(
    pl.kernel,
    out_type=jax.ShapeDtypeStruct((NC, ACC_ROWS), jnp.float32),
    mesh=_mesh,
    scratch_types=[
        pltpu.VMEM((ROWS_PER_TILE, 128), jnp.int32),
        pltpu.VMEM((128,), jnp.float32),
        pltpu.VMEM((640,), jnp.float32),
        pltpu.VMEM_SHARED((ACC_ROWS,), jnp.float32),
        pltpu.SemaphoreType.DMA,
    ],
)
def _sc_degree(dst_hbm, out_hbm, didx_all, ones_v, z_v, acc_sh, sem):
    cid = lax.axis_index("c")
    sid = lax.axis_index("s")
    wid = _worker_id()

    pltpu.sync_copy(dst_hbm.at[pl.ds(wid * ROWS_PER_TILE, ROWS_PER_TILE)],
                    didx_all)

    one16 = jnp.ones((16,), jnp.float32)
    zero16 = jnp.zeros((16,), jnp.float32)
    for j in range(8):
        ones_v[pl.ds(j * 16, 16)] = one16
    for j in range(40):
        z_v[pl.ds(j * 16, 16)] = zero16

    pltpu.sync_copy(z_v, acc_sh.at[pl.ds(sid * 640, 640)])
    plsc.subcore_barrier()

    def blk(g, carry):
        for b in range(8):
            c = g * 8 + b
            pltpu.async_copy(ones_v, acc_sh.at[didx_all.at[c]], sem,
                             add=True)
        for b in range(8):
            pltpu.make_async_copy(ones_v, acc_sh.at[didx_all.at[0]],
                                  sem).wait()
        return carry

    lax.fori_loop(0, ROWS_PER_TILE // 8, blk, 0)
    plsc.subcore_barrier()
    pltpu.sync_copy(acc_sh.at[pl.ds(sid * 640, 640)],
                    out_hbm.at[cid, pl.ds(sid * 640, 640)])


_NBUF = 2


@functools.partial(
    pl.kernel,
    out_type=jax.ShapeDtypeStruct((NC, ACC_ROWS, D), jnp.float32),
    mesh=_mesh,
    scratch_types=(
        [
            pltpu.VMEM((ROWS_PER_TILE, 128), jnp.int32),
            pltpu.VMEM((8, 128), jnp.int32),
        ]
        + [pltpu.VMEM((128, D), jnp.float32)] * _NBUF
        + [pltpu.VMEM_SHARED((ACC_ROWS, D), jnp.float32)]
        + [pltpu.SemaphoreType.DMA] * (2 * _NBUF)
    ),
)
def _sc_aggregate(h_hbm, src_hbm, dst_hbm, out_hbm,
                  sidx_all, didx_v, *rest):
    rows = rest[:_NBUF]
    acc_sh = rest[_NBUF]
    gsem = rest[_NBUF + 1:_NBUF + 1 + _NBUF]
    ssem = rest[_NBUF + 1 + _NBUF:]

    cid = lax.axis_index("c")
    sid = lax.axis_index("s")
    wid = _worker_id()

    pltpu.sync_copy(src_hbm.at[pl.ds(wid * ROWS_PER_TILE, ROWS_PER_TILE)],
                    sidx_all)

    zero16 = jnp.zeros((16,), jnp.float32)

    def zrow(r, carry):
        for c in range(8):
            rows[0][r, pl.ds(c * 16, 16)] = zero16
        return carry

    lax.fori_loop(0, 128, zrow, 0)

    def zcp(t, carry):
        pltpu.sync_copy(rows[0], acc_sh.at[pl.ds(sid * 640 + t * 128, 128)])
        return carry

    lax.fori_loop(0, 5, zcp, 0)

    for b in range(_NBUF):
        pltpu.async_copy(h_hbm.at[sidx_all.at[b]], rows[b], gsem[b])
    plsc.subcore_barrier()

    def blk(g, carry):
        pltpu.sync_copy(dst_hbm.at[pl.ds(wid * ROWS_PER_TILE + g * 8, 8)],
                        didx_v)
        for b8 in range(8):
            c = g * 8 + b8
            b = b8 % _NBUF
            pltpu.make_async_copy(h_hbm.at[sidx_all.at[c]], rows[b],
                                  gsem[b]).wait()
            pltpu.async_copy(rows[b], acc_sh.at[didx_v.at[b8]], ssem[b],
                             add=True)

            @pl.when(c + _NBUF < ROWS_PER_TILE)
            def _():
                pltpu.make_async_copy(rows[b], acc_sh.at[didx_v.at[b8]],
                                      ssem[b]).wait()
                pltpu.async_copy(h_hbm.at[sidx_all.at[c + _NBUF]], rows[b],
                                 gsem[b])
        return carry

    lax.fori_loop(0, ROWS_PER_TILE // 8, blk, 0)
    for b in range(_NBUF):
        pltpu.make_async_copy(rows[b], acc_sh.at[didx_v.at[b]],
                              ssem[b]).wait()
    plsc.subcore_barrier()
    pltpu.sync_copy(acc_sh.at[pl.ds(sid * 640, 640)],
                    out_hbm.at[cid, pl.ds(sid * 640, 640)])


_GRID = 10
_BR = N_NODES // _GRID


def _dis_of(degp_ref):
    deg = degp_ref[:, 0] + degp_ref[:, 1] + 1.0
    return lax.rsqrt(deg)


def _tc1_body(x_ref, w_ref, degp_ref, o_ref):
    dis = _dis_of(degp_ref)
    h = jnp.dot(x_ref[...], w_ref[...], preferred_element_type=jnp.float32)
    o_ref[...] = h * dis[:, None]


def _tc2_body(agg_ref, hp_ref, degp_ref, b_ref, w_ref, o_ref):
    dis = _dis_of(degp_ref)
    t = (agg_ref[0] + agg_ref[1] + hp_ref[...]) * dis[:, None] + b_ref[...]
    t = jnp.maximum(t, 0.0)
    h = jnp.dot(t, w_ref[...], preferred_element_type=jnp.float32)
    o_ref[...] = h * dis[:, None]


def _tc3_body(agg_ref, hp_ref, degp_ref, b_ref, o_ref):
    dis = _dis_of(degp_ref)
    o_ref[...] = ((agg_ref[0] + agg_ref[1] + hp_ref[...]) * dis[:, None]
                  + b_ref[...])


_ROWS_SPEC = pl.BlockSpec((_BR, D), lambda i: (i, 0))
_W_SPEC = pl.BlockSpec((D, D), lambda i: (0, 0))
_DEG_SPEC = pl.BlockSpec((_BR, NC), lambda i: (i, 0))
_AGG_SPEC = pl.BlockSpec((NC, _BR, D), lambda i: (0, i, 0))
_B_SPEC = pl.BlockSpec((1, D), lambda i: (0, 0))

_tc1 = pl.pallas_call(
    _tc1_body,
    grid=(_GRID,),
    in_specs=[_ROWS_SPEC, _W_SPEC, _DEG_SPEC],
    out_specs=_ROWS_SPEC,
    out_shape=jax.ShapeDtypeStruct((N_NODES, D), jnp.float32),
)

_tc2 = pl.pallas_call(
    _tc2_body,
    grid=(_GRID,),
    in_specs=[_AGG_SPEC, _ROWS_SPEC, _DEG_SPEC, _B_SPEC, _W_SPEC],
    out_specs=_ROWS_SPEC,
    out_shape=jax.ShapeDtypeStruct((N_NODES, D), jnp.float32),
)

_tc3 = pl.pallas_call(
    _tc3_body,
    grid=(_GRID,),
    in_specs=[_AGG_SPEC, _ROWS_SPEC, _DEG_SPEC, _B_SPEC],
    out_specs=_ROWS_SPEC,
    out_shape=jax.ShapeDtypeStruct((N_NODES, D), jnp.float32),
)


def _pad_h(h):
    return jnp.concatenate([h, jnp.zeros((H_ROWS - N_NODES, D), jnp.float32)],
                           axis=0)


def kernel(x, edge_index, W1, b1, W2, b2):
    src = edge_index[0].astype(jnp.int32)
    dst = edge_index[1].astype(jnp.int32)

    npad = E_PAD - N_EDGES
    pad_i = jnp.arange(npad, dtype=jnp.int32)
    src_p = jnp.concatenate([src, N_NODES + pad_i % (H_ROWS - N_NODES)])
    dst_p = jnp.concatenate([dst, N_NODES + pad_i % (ACC_ROWS - N_NODES)])
    src2d = src_p.reshape(IDX_ROWS, 128)
    dst2d = dst_p.reshape(IDX_ROWS, 128)

    b1r = b1.reshape(1, D)
    b2r = b2.reshape(1, D)

    degp = _sc_degree(dst2d).T
    h1 = _tc1(x, W1, degp)
    agg1 = _sc_aggregate(_pad_h(h1), src2d, dst2d)
    h2 = _tc2(agg1, h1, degp, b1r, W2)
    agg2 = _sc_aggregate(_pad_h(h2), src2d, dst2d)
    return _tc3(agg2, h2, degp, b2r)


# speedup vs baseline: 34.0513x; 1.0334x over previous
import functools

import jax
import jax.numpy as jnp
from jax import lax
from jax.experimental import pallas as pl
from jax.experimental.pallas import tpu as pltpu
from jax.experimental.pallas import tpu_sc as plsc

N_NODES = 10000
N_EDGES = 320000
D = 128

NC = 2
NS = 16
NW = NC * NS

E_PER_TILE = 10240
E_PAD = E_PER_TILE * NW
IDX_ROWS = E_PAD // 128
ROWS_PER_TILE = IDX_ROWS // NW
BLOCKS_PER_TILE = ROWS_PER_TILE // 8

ACC_ROWS = 10240

_mesh = plsc.VectorSubcoreMesh(core_axis_name="c", subcore_axis_name="s")


def _worker_id():
    return lax.axis_index("c") * NS + lax.axis_index("s")


@functools.partial(
    pl.kernel,
    out_type=jax.ShapeDtypeStruct((NC, ACC_ROWS), jnp.float32),
    mesh=_mesh,
    scratch_types=[
        pltpu.VMEM((ROWS_PER_TILE, 128), jnp.int32),
        pltpu.VMEM((128,), jnp.float32),
        pltpu.VMEM((640,), jnp.float32),
        pltpu.VMEM_SHARED((ACC_ROWS,), jnp.float32),
        pltpu.SemaphoreType.DMA,
    ],
)
def _sc_degree(dst_hbm, out_hbm, didx_all, ones_v, z_v, acc_sh, sem):
    cid = lax.axis_index("c")
    sid = lax.axis_index("s")
    wid = _worker_id()

    pltpu.sync_copy(dst_hbm.at[pl.ds(wid * ROWS_PER_TILE, ROWS_PER_TILE)],
                    didx_all)

    one16 = jnp.ones((16,), jnp.float32)
    zero16 = jnp.zeros((16,), jnp.float32)
    for j in range(8):
        ones_v[pl.ds(j * 16, 16)] = one16
    for j in range(40):
        z_v[pl.ds(j * 16, 16)] = zero16

    pltpu.sync_copy(z_v, acc_sh.at[pl.ds(sid * 640, 640)])
    plsc.subcore_barrier()

    def blk(g, carry):
        for b in range(8):
            c = g * 8 + b
            pltpu.async_copy(ones_v, acc_sh.at[didx_all.at[c]], sem,
                             add=True)
        for b in range(8):
            pltpu.make_async_copy(ones_v, acc_sh.at[didx_all.at[0]],
                                  sem).wait()
        return carry

    lax.fori_loop(0, ROWS_PER_TILE // 8, blk, 0)
    plsc.subcore_barrier()
    pltpu.sync_copy(acc_sh.at[pl.ds(sid * 640, 640)],
                    out_hbm.at[cid, pl.ds(sid * 640, 640)])


_NBUF = 2


@functools.partial(
    pl.kernel,
    out_type=jax.ShapeDtypeStruct((NC, ACC_ROWS, D), jnp.float32),
    mesh=_mesh,
    scratch_types=(
        [
            pltpu.VMEM((ROWS_PER_TILE, 128), jnp.int32),
            pltpu.VMEM((8, 128), jnp.int32),
        ]
        + [pltpu.VMEM((128, D), jnp.float32)] * _NBUF
        + [pltpu.VMEM_SHARED((ACC_ROWS, D), jnp.float32)]
        + [pltpu.SemaphoreType.DMA] * (2 * _NBUF)
    ),
)
def _sc_aggregate(h_hbm, src_hbm, dst_hbm, out_hbm,
                  sidx_all, didx_v, *rest):
    rows = rest[:_NBUF]
    acc_sh = rest[_NBUF]
    gsem = rest[_NBUF + 1:_NBUF + 1 + _NBUF]
    ssem = rest[_NBUF + 1 + _NBUF:]

    cid = lax.axis_index("c")
    sid = lax.axis_index("s")
    wid = _worker_id()

    pltpu.sync_copy(src_hbm.at[pl.ds(wid * ROWS_PER_TILE, ROWS_PER_TILE)],
                    sidx_all)

    zero16 = jnp.zeros((16,), jnp.float32)

    def zrow(r, carry):
        for c in range(8):
            rows[0][r, pl.ds(c * 16, 16)] = zero16
        return carry

    lax.fori_loop(0, 128, zrow, 0)

    def zcp(t, carry):
        pltpu.sync_copy(rows[0], acc_sh.at[pl.ds(sid * 640 + t * 128, 128)])
        return carry

    lax.fori_loop(0, 5, zcp, 0)

    for b in range(_NBUF):
        pltpu.async_copy(h_hbm.at[sidx_all.at[b]], rows[b], gsem[b])
    plsc.subcore_barrier()

    def blk(g, carry):
        pltpu.sync_copy(dst_hbm.at[pl.ds(wid * ROWS_PER_TILE + g * 8, 8)],
                        didx_v)
        for b8 in range(8):
            c = g * 8 + b8
            b = b8 % _NBUF
            pltpu.make_async_copy(h_hbm.at[sidx_all.at[c]], rows[b],
                                  gsem[b]).wait()
            pltpu.async_copy(rows[b], acc_sh.at[didx_v.at[b8]], ssem[b],
                             add=True)

            @pl.when(c + _NBUF < ROWS_PER_TILE)
            def _():
                pltpu.make_async_copy(rows[b], acc_sh.at[didx_v.at[b8]],
                                      ssem[b]).wait()
                pltpu.async_copy(h_hbm.at[sidx_all.at[c + _NBUF]], rows[b],
                                 gsem[b])
        return carry

    lax.fori_loop(0, ROWS_PER_TILE // 8, blk, 0)
    for b in range(_NBUF):
        pltpu.make_async_copy(rows[b], acc_sh.at[didx_v.at[b]],
                              ssem[b]).wait()
    plsc.subcore_barrier()
    pltpu.sync_copy(acc_sh.at[pl.ds(sid * 640, 640)],
                    out_hbm.at[cid, pl.ds(sid * 640, 640)])


_GRID = 10
_BR = N_NODES // _GRID


def _dis_of(degp_ref):
    deg = degp_ref[:, 0] + degp_ref[:, 1] + 1.0
    return lax.rsqrt(deg)


def _tc1_body(x_ref, w_ref, degp_ref, o_ref):
    dis = _dis_of(degp_ref)
    h = jnp.dot(x_ref[...], w_ref[...], preferred_element_type=jnp.float32)
    o_ref[...] = h * dis[:, None]


def _tc2_body(agg_ref, hp_ref, degp_ref, b_ref, w_ref, o_ref):
    dis = _dis_of(degp_ref)
    t = (agg_ref[0] + agg_ref[1] + hp_ref[...]) * dis[:, None] + b_ref[...]
    t = jnp.maximum(t, 0.0)
    h = jnp.dot(t, w_ref[...], preferred_element_type=jnp.float32)
    o_ref[...] = h * dis[:, None]


def _tc3_body(agg_ref, hp_ref, degp_ref, b_ref, o_ref):
    dis = _dis_of(degp_ref)
    o_ref[...] = ((agg_ref[0] + agg_ref[1] + hp_ref[...]) * dis[:, None]
                  + b_ref[...])


_ROWS_SPEC = pl.BlockSpec((_BR, D), lambda i: (i, 0))
_W_SPEC = pl.BlockSpec((D, D), lambda i: (0, 0))
_DEG_SPEC = pl.BlockSpec((_BR, NC), lambda i: (i, 0))
_AGG_SPEC = pl.BlockSpec((NC, _BR, D), lambda i: (0, i, 0))
_B_SPEC = pl.BlockSpec((1, D), lambda i: (0, 0))

_tc1 = pl.pallas_call(
    _tc1_body,
    grid=(_GRID,),
    in_specs=[_ROWS_SPEC, _W_SPEC, _DEG_SPEC],
    out_specs=_ROWS_SPEC,
    out_shape=jax.ShapeDtypeStruct((N_NODES, D), jnp.float32),
)

_tc2 = pl.pallas_call(
    _tc2_body,
    grid=(_GRID,),
    in_specs=[_AGG_SPEC, _ROWS_SPEC, _DEG_SPEC, _B_SPEC, _W_SPEC],
    out_specs=_ROWS_SPEC,
    out_shape=jax.ShapeDtypeStruct((N_NODES, D), jnp.float32),
)

_tc3 = pl.pallas_call(
    _tc3_body,
    grid=(_GRID,),
    in_specs=[_AGG_SPEC, _ROWS_SPEC, _DEG_SPEC, _B_SPEC],
    out_specs=_ROWS_SPEC,
    out_shape=jax.ShapeDtypeStruct((N_NODES, D), jnp.float32),
)


def kernel(x, edge_index, W1, b1, W2, b2):
    src = edge_index[0].astype(jnp.int32)
    dst = edge_index[1].astype(jnp.int32)

    npad = E_PAD - N_EDGES
    pad_i = jnp.arange(npad, dtype=jnp.int32)
    src_p = jnp.concatenate([src, pad_i % N_NODES])
    dst_p = jnp.concatenate([dst, N_NODES + pad_i % (ACC_ROWS - N_NODES)])
    src2d = src_p.reshape(IDX_ROWS, 128)
    dst2d = dst_p.reshape(IDX_ROWS, 128)

    b1r = b1.reshape(1, D)
    b2r = b2.reshape(1, D)

    degp = _sc_degree(dst2d).T
    h1 = _tc1(x, W1, degp)
    agg1 = _sc_aggregate(h1, src2d, dst2d)
    h2 = _tc2(agg1, h1, degp, b1r, W2)
    agg2 = _sc_aggregate(h2, src2d, dst2d)
    return _tc3(agg2, h2, degp, b2r)


# speedup vs baseline: 34.2198x; 1.0049x over previous
import functools

import jax
import jax.numpy as jnp
from jax import lax
from jax.experimental import pallas as pl
from jax.experimental.pallas import tpu as pltpu
from jax.experimental.pallas import tpu_sc as plsc

N_NODES = 10000
N_EDGES = 320000
D = 128

NC = 2
NS = 16
NW = NC * NS

E_PER_TILE = 10240
E_PAD = E_PER_TILE * NW
IDX_ROWS = E_PAD // 128
ROWS_PER_TILE = IDX_ROWS // NW

ACC_ROWS = 10240

_mesh = plsc.VectorSubcoreMesh(core_axis_name="c", subcore_axis_name="s")


def _worker_id():
    return lax.axis_index("c") * NS + lax.axis_index("s")


def _extract_row(pk_all, c, dst_ring=None, b=0, src_ring=None):
    mask = jnp.full((16,), 0xFFFF, jnp.int32)
    for k in range(8):
        v = pk_all[c, pl.ds(k * 16, 16)]
        if src_ring is not None:
            src_ring[b, pl.ds(k * 16, 16)] = jnp.bitwise_and(v, mask)
        if dst_ring is not None:
            dst_ring[b, pl.ds(k * 16, 16)] = jnp.right_shift(v, 16)


@functools.partial(
    pl.kernel,
    out_type=jax.ShapeDtypeStruct((NC, ACC_ROWS), jnp.float32),
    mesh=_mesh,
    scratch_types=[
        pltpu.VMEM((ROWS_PER_TILE, 128), jnp.int32),
        pltpu.VMEM((8, 128), jnp.int32),
        pltpu.VMEM((128,), jnp.float32),
        pltpu.VMEM((640,), jnp.float32),
        pltpu.VMEM_SHARED((ACC_ROWS,), jnp.float32),
        pltpu.SemaphoreType.DMA,
    ],
)
def _sc_degree(eidx_hbm, out_hbm, pk_all, ring, ones_v, z_v, acc_sh, sem):
    cid = lax.axis_index("c")
    sid = lax.axis_index("s")
    wid = _worker_id()

    pltpu.sync_copy(eidx_hbm.at[pl.ds(wid * ROWS_PER_TILE, ROWS_PER_TILE)],
                    pk_all)

    one16 = jnp.ones((16,), jnp.float32)
    zero16 = jnp.zeros((16,), jnp.float32)
    for j in range(8):
        ones_v[pl.ds(j * 16, 16)] = one16
    for j in range(40):
        z_v[pl.ds(j * 16, 16)] = zero16

    pltpu.sync_copy(z_v, acc_sh.at[pl.ds(sid * 640, 640)])
    plsc.subcore_barrier()

    def blk(g, carry):
        for b in range(8):
            _extract_row(pk_all, g * 8 + b, dst_ring=ring, b=b)
        for b in range(8):
            pltpu.async_copy(ones_v, acc_sh.at[ring.at[b]], sem, add=True)
        for b in range(8):
            pltpu.make_async_copy(ones_v, acc_sh.at[ring.at[0]], sem).wait()
        return carry

    lax.fori_loop(0, ROWS_PER_TILE // 8, blk, 0)
    plsc.subcore_barrier()
    pltpu.sync_copy(acc_sh.at[pl.ds(sid * 640, 640)],
                    out_hbm.at[cid, pl.ds(sid * 640, 640)])


_NBUF = 2


@functools.partial(
    pl.kernel,
    out_type=jax.ShapeDtypeStruct((NC, ACC_ROWS, D), jnp.float32),
    mesh=_mesh,
    scratch_types=(
        [
            pltpu.VMEM((ROWS_PER_TILE, 128), jnp.int32),
            pltpu.VMEM((_NBUF, 128), jnp.int32),
            pltpu.VMEM((_NBUF, 128), jnp.int32),
        ]
        + [pltpu.VMEM((128, D), jnp.float32)] * _NBUF
        + [
            pltpu.VMEM((16, D), jnp.float32),
            pltpu.VMEM_SHARED((ACC_ROWS, D), jnp.float32),
        ]
        + [pltpu.SemaphoreType.DMA] * (2 * _NBUF)
    ),
)
def _sc_aggregate(h_hbm, eidx_hbm, out_hbm, pk_all, sring, dring, *rest):
    rows = rest[:_NBUF]
    z_v = rest[_NBUF]
    acc_sh = rest[_NBUF + 1]
    gsem = rest[_NBUF + 2:_NBUF + 2 + _NBUF]
    ssem = rest[_NBUF + 2 + _NBUF:]

    cid = lax.axis_index("c")
    sid = lax.axis_index("s")
    wid = _worker_id()

    pltpu.sync_copy(eidx_hbm.at[pl.ds(wid * ROWS_PER_TILE, ROWS_PER_TILE)],
                    pk_all)

    for b in range(_NBUF):
        _extract_row(pk_all, b, dst_ring=dring, b=b, src_ring=sring)
        pltpu.async_copy(h_hbm.at[sring.at[b]], rows[b], gsem[b])

    zero16 = jnp.zeros((16,), jnp.float32)
    for r in range(16):
        for c in range(8):
            z_v[r, pl.ds(c * 16, 16)] = zero16

    def zcp(t, carry):
        pltpu.sync_copy(z_v, acc_sh.at[pl.ds(sid * 640 + t * 16, 16)])
        return carry

    lax.fori_loop(0, 40, zcp, 0)
    plsc.subcore_barrier()

    def blk(g, carry):
        for b in range(_NBUF):
            c = g * _NBUF + b
            pltpu.make_async_copy(h_hbm.at[sring.at[b]], rows[b],
                                  gsem[b]).wait()
            pltpu.async_copy(rows[b], acc_sh.at[dring.at[b]], ssem[b],
                             add=True)

            @pl.when(c + _NBUF < ROWS_PER_TILE)
            def _():
                pltpu.make_async_copy(rows[b], acc_sh.at[dring.at[b]],
                                      ssem[b]).wait()
                _extract_row(pk_all, c + _NBUF, dst_ring=dring, b=b,
                             src_ring=sring)
                pltpu.async_copy(h_hbm.at[sring.at[b]], rows[b], gsem[b])
        return carry

    lax.fori_loop(0, ROWS_PER_TILE // _NBUF, blk, 0)
    for b in range(_NBUF):
        pltpu.make_async_copy(rows[b], acc_sh.at[dring.at[b]],
                              ssem[b]).wait()
    plsc.subcore_barrier()
    pltpu.sync_copy(acc_sh.at[pl.ds(sid * 640, 640)],
                    out_hbm.at[cid, pl.ds(sid * 640, 640)])


_GRID = 10
_BR = N_NODES // _GRID


def _dis_of(degp_ref):
    deg = degp_ref[:, 0] + degp_ref[:, 1] + 1.0
    return lax.rsqrt(deg)


def _tc1_body(x_ref, w_ref, degp_ref, o_ref):
    dis = _dis_of(degp_ref)
    h = jnp.dot(x_ref[...], w_ref[...], preferred_element_type=jnp.float32)
    o_ref[...] = h * dis[:, None]


def _tc2_body(agg_ref, hp_ref, degp_ref, b_ref, w_ref, o_ref):
    dis = _dis_of(degp_ref)
    t = (agg_ref[0] + agg_ref[1] + hp_ref[...]) * dis[:, None] + b_ref[...]
    t = jnp.maximum(t, 0.0)
    h = jnp.dot(t, w_ref[...], preferred_element_type=jnp.float32)
    o_ref[...] = h * dis[:, None]


def _tc3_body(agg_ref, hp_ref, degp_ref, b_ref, o_ref):
    dis = _dis_of(degp_ref)
    o_ref[...] = ((agg_ref[0] + agg_ref[1] + hp_ref[...]) * dis[:, None]
                  + b_ref[...])


_ROWS_SPEC = pl.BlockSpec((_BR, D), lambda i: (i, 0))
_W_SPEC = pl.BlockSpec((D, D), lambda i: (0, 0))
_DEG_SPEC = pl.BlockSpec((_BR, NC), lambda i: (i, 0))
_AGG_SPEC = pl.BlockSpec((NC, _BR, D), lambda i: (0, i, 0))
_B_SPEC = pl.BlockSpec((1, D), lambda i: (0, 0))

_tc1 = pl.pallas_call(
    _tc1_body,
    grid=(_GRID,),
    in_specs=[_ROWS_SPEC, _W_SPEC, _DEG_SPEC],
    out_specs=_ROWS_SPEC,
    out_shape=jax.ShapeDtypeStruct((N_NODES, D), jnp.float32),
)

_tc2 = pl.pallas_call(
    _tc2_body,
    grid=(_GRID,),
    in_specs=[_AGG_SPEC, _ROWS_SPEC, _DEG_SPEC, _B_SPEC, _W_SPEC],
    out_specs=_ROWS_SPEC,
    out_shape=jax.ShapeDtypeStruct((N_NODES, D), jnp.float32),
)

_tc3 = pl.pallas_call(
    _tc3_body,
    grid=(_GRID,),
    in_specs=[_AGG_SPEC, _ROWS_SPEC, _DEG_SPEC, _B_SPEC],
    out_specs=_ROWS_SPEC,
    out_shape=jax.ShapeDtypeStruct((N_NODES, D), jnp.float32),
)


def kernel(x, edge_index, W1, b1, W2, b2):
    src = edge_index[0].astype(jnp.int32)
    dst = edge_index[1].astype(jnp.int32)

    npad = E_PAD - N_EDGES
    pad_i = jnp.arange(npad, dtype=jnp.int32)
    pad_pk = (pad_i % N_NODES) | (
        (N_NODES + pad_i % (ACC_ROWS - N_NODES)) << 16)
    eidx = jnp.concatenate([src | (dst << 16), pad_pk]).reshape(IDX_ROWS, 128)

    b1r = b1.reshape(1, D)
    b2r = b2.reshape(1, D)

    degp = _sc_degree(eidx).T
    h1 = _tc1(x, W1, degp)
    agg1 = _sc_aggregate(h1, eidx)
    h2 = _tc2(agg1, h1, degp, b1r, W2)
    agg2 = _sc_aggregate(h2, eidx)
    return _tc3(agg2, h2, degp, b2r)
